# Initial kernel scaffold; baseline (speedup 1.0000x reference)
import jax
import jax.numpy as jnp
from jax.experimental import pallas as pl


def kernel(x, edge_index, labels, W0, U0, c0, b0, W1, U1, c1, b1, Wu, Uu, cu, bu, p0):
    raise NotImplementedError("write your pallas kernel here")


# speedup vs baseline: 1.3728x; 1.3728x over previous
import functools
import math

import jax
import jax.numpy as jnp
from jax.experimental import pallas as pl
from jax.experimental.pallas import tpu as pltpu

N = 10000
E = 320000
D = 128
K = 5000


def _t1_body(agg0_ref, x_ref, deg_ref, w0_ref, b0_ref, p0n_ref, wu_ref,
             x0_ref, score_ref, y0_ref):
    a = (agg0_ref[...] + x_ref[...])
    x0 = jnp.maximum(jnp.dot(a, w0_ref[...], preferred_element_type=jnp.float32)
                     / deg_ref[...] + b0_ref[...], 0.0)
    x0_ref[...] = x0
    score_ref[...] = jnp.dot(x0, p0n_ref[...], preferred_element_type=jnp.float32)
    y0_ref[...] = jnp.dot(x0, wu_ref[...], preferred_element_type=jnp.float32)


def _t1(agg0, x, deg, W0, b0, p0n, Wu):
    return pl.pallas_call(
        _t1_body,
        out_shape=(
            jax.ShapeDtypeStruct((N, D), jnp.float32),
            jax.ShapeDtypeStruct((N, 1), jnp.float32),
            jax.ShapeDtypeStruct((N, 1), jnp.float32),
        ),
    )(agg0, x, deg, W0, b0, p0n, Wu)


KP = 5120
BI = 200
BJ = 640


def _t2_body(atr_ref, ar_ref, m_ref, deg1_ref):
    j = pl.program_id(1)
    p = jnp.dot(atr_ref[...], ar_ref[...].T, preferred_element_type=jnp.float32)
    m = (p > 0.0).astype(jnp.float32)
    m_ref[...] = m.astype(jnp.bfloat16)

    @pl.when(j == 0)
    def _():
        deg1_ref[...] = jnp.zeros_like(deg1_ref)

    deg1_ref[...] += jnp.sum(m, axis=1, keepdims=True)


def _t2(ATr, Ar_pad):
    grid = (K // BI, KP // BJ)
    return pl.pallas_call(
        _t2_body,
        grid=grid,
        in_specs=[
            pl.BlockSpec((BI, N), lambda i, j: (i, 0)),
            pl.BlockSpec((BJ, N), lambda i, j: (j, 0)),
        ],
        out_specs=(
            pl.BlockSpec((BI, BJ), lambda i, j: (i, j)),
            pl.BlockSpec((BI, 1), lambda i, j: (i, 0)),
        ),
        out_shape=(
            jax.ShapeDtypeStruct((K, KP), jnp.bfloat16),
            jax.ShapeDtypeStruct((K, 1), jnp.float32),
        ),
        compiler_params=pltpu.CompilerParams(
            dimension_semantics=("parallel", "arbitrary")),
    )(ATr, Ar_pad)


def _t3_body(xpr_ref, sp_ref, w1_ref, out_ref):
    xp = xpr_ref[...] * jnp.tanh(sp_ref[...])
    out_ref[...] = jnp.dot(xp, w1_ref[...], preferred_element_type=jnp.float32)


def _t3(xp_rows, sp, W1):
    return pl.pallas_call(
        _t3_body,
        out_shape=jax.ShapeDtypeStruct((K, D), jnp.float32),
    )(xp_rows, sp, W1)


BI4 = 200


def _t4_body(m_ref, xw1_ref, deg1_ref, b1_ref, wu_ref, y1_ref):
    agg = jnp.dot(m_ref[...].astype(jnp.float32), xw1_ref[...],
                  preferred_element_type=jnp.float32)
    x1 = jnp.maximum(agg / jnp.maximum(deg1_ref[...], 1.0) + b1_ref[...], 0.0)
    y1_ref[...] = jnp.dot(x1, wu_ref[...], preferred_element_type=jnp.float32)


def _t4(M, xw1, deg1, b1, Wu):
    grid = (K // BI4,)
    return pl.pallas_call(
        _t4_body,
        grid=grid,
        in_specs=[
            pl.BlockSpec((BI4, KP), lambda i: (i, 0)),
            pl.BlockSpec((KP, D), lambda i: (0, 0)),
            pl.BlockSpec((BI4, 1), lambda i: (i, 0)),
            pl.BlockSpec((1, D), lambda i: (0, 0)),
            pl.BlockSpec((D, 1), lambda i: (0, 0)),
        ],
        out_specs=pl.BlockSpec((BI4, 1), lambda i: (i, 0)),
        out_shape=jax.ShapeDtypeStruct((K, 1), jnp.float32),
    )(M, xw1, deg1, b1, Wu)


def _t6_body(aggu_ref, y_ref, deg_ref, bu_ref, lab_ref, xo_ref, loss_ref):
    xo = (aggu_ref[...] + y_ref[...]) / deg_ref[...] + bu_ref[...]
    xo_ref[...] = xo
    xcl = jnp.clip(xo, 1e-7, 1.0 - 1e-7)
    lab = lab_ref[...]
    pos = jnp.sum(lab)
    neg = lab.size - pos
    wgt = lab * (neg / jnp.maximum(pos, 1.0)) + (1.0 - lab)
    bce = -(lab * jnp.log(xcl) + (1.0 - lab) * jnp.log(1.0 - xcl))
    loss_ref[...] = jnp.reshape(jnp.sum(wgt * bce) / lab.size, (1, 1))


def _t6(aggu, y, deg, bu, labels):
    return pl.pallas_call(
        _t6_body,
        out_shape=(
            jax.ShapeDtypeStruct((N, 1), jnp.float32),
            jax.ShapeDtypeStruct((1, 1), jnp.float32),
        ),
    )(aggu, y, deg, bu, labels)


def kernel(x, edge_index, labels, W0, U0, c0, b0, W1, U1, c1, b1, Wu, Uu, cu, bu, p0):
    src = edge_index[0]
    dst = edge_index[1]

    agg0 = jax.ops.segment_sum(x[src], dst, num_segments=N)
    deg = (jax.ops.segment_sum(jnp.ones((E,), jnp.float32), dst, num_segments=N)
           + 1.0)[:, None]

    p0n = (p0 / jnp.linalg.norm(p0))[:, None]
    x0, score2, y0 = _t1(agg0, x, deg, W0, b0[None, :], p0n, Wu)
    score = score2[:, 0]

    perm = jax.lax.top_k(score, K)[1]

    A = jnp.zeros((N, N), jnp.float32).at[src, dst].set(1.0)
    diag = jnp.arange(N)
    A = A.at[diag, diag].set(1.0)
    Ar = A[perm].astype(jnp.bfloat16)
    Ar_pad = jnp.concatenate([Ar, jnp.zeros((KP - K, N), jnp.bfloat16)], axis=0)
    ATr = A.T[perm].astype(jnp.bfloat16)

    M, deg1 = _t2(ATr, Ar_pad)

    xp_rows = x0[perm]
    sp = score[perm][:, None]
    xw1 = _t3(xp_rows, sp, W1)
    xw1_pad = jnp.concatenate([xw1, jnp.zeros((KP - K, D), jnp.float32)], axis=0)
    y1 = _t4(M, xw1_pad, deg1, b1[None, :], Wu)

    y = y0.at[perm, 0].add(y1[:, 0])
    aggu = jax.ops.segment_sum(y[src, 0], dst, num_segments=N)[:, None]
    xo, loss = _t6(aggu, y, deg, bu, labels)
    return (loss[0, 0], xo)


# speedup vs baseline: 1.6532x; 1.2042x over previous
import functools
import math

import jax
import jax.numpy as jnp
from jax import lax
from jax.experimental import pallas as pl
from jax.experimental.pallas import tpu as pltpu
from jax.experimental.pallas import tpu_sc as plsc

N = 10000
E = 320000
D = 128
K = 5000

NP = 10240
KR = 5120
LP = KR * NP
E_PAD = 327680
EPT = E_PAD // 16
ZSPAN = LP // 16
ZB = 16384
ZITERS = ZSPAN // ZB


def _s2_body(se_hbm, de_hbm, rank_hbm, ar_hbm, atr_hbm,
             zeros_v, ones_v, se_v, de_v,
             i0, i1, i2, i3, i4, i5, i6, i7,
             r0, r1, r2, r3, r4, r5, r6, r7, sem, sem2):
    c_ax = lax.axis_index("c")
    s_ax = lax.axis_index("s")
    idxrs = (i0, i1, i2, i3, i4, i5, i6, i7)
    rks = (r0, r1, r2, r3, r4, r5, r6, r7)

    def init_bufs(i, _):
        zeros_v[pl.ds(i * 16, 16)] = jnp.zeros((16,), jnp.float32)
        return 0

    lax.fori_loop(0, ZB // 16, init_bufs, 0)
    for j in range(8):
        ones_v[pl.ds(j * 16, 16)] = jnp.ones((16,), jnp.float32)

    def do_zero(out_ref):
        zb = s_ax * ZSPAN

        def zstep(i, _):
            pltpu.sync_copy(zeros_v, out_ref.at[pl.ds(zb + i * ZB, ZB)])
            return 0

        lax.fori_loop(0, ZITERS, zstep, 0)

    def do_scatter(out_ref, is_ar):
        tbase = s_ax * EPT
        key_v = se_v if is_ar else de_v
        col_v = de_v if is_ar else se_v

        def ostep(o, _):
            pltpu.sync_copy(se_hbm.at[pl.ds(tbase + o * 1024, 1024)], se_v)
            pltpu.sync_copy(de_hbm.at[pl.ds(tbase + o * 1024, 1024)], de_v)
            gathers = [
                pltpu.async_copy(
                    rank_hbm.at[key_v.at[pl.ds(r * 128, 128)]], rks[r], sem2)
                for r in range(8)
            ]
            for cp in gathers:
                cp.wait()
            for r in range(8):
                for j in range(8):
                    off = r * 128 + j * 16
                    rk16 = rks[r][pl.ds(j * 16, 16)]
                    col16 = col_v[pl.ds(off, 16)]
                    idxrs[r][pl.ds(j * 16, 16)] = rk16 * NP + col16
            scatters = [
                pltpu.async_copy(ones_v, out_ref.at[idxrs[r]], sem)
                for r in range(8)
            ]
            for cp in scatters:
                cp.wait()
            return 0

        lax.fori_loop(0, EPT // 1024, ostep, 0)

    @pl.when(c_ax == 0)
    def _():
        do_zero(ar_hbm)

    @pl.when(c_ax == 1)
    def _():
        do_zero(atr_hbm)

    plsc.subcore_barrier()

    @pl.when(c_ax == 0)
    def _():
        do_scatter(ar_hbm, True)

    @pl.when(c_ax == 1)
    def _():
        do_scatter(atr_hbm, False)


def _s2(se, de, rank_ext):
    mesh = plsc.VectorSubcoreMesh(core_axis_name="c", subcore_axis_name="s")
    f = functools.partial(
        pl.kernel, mesh=mesh,
        out_type=(
            jax.ShapeDtypeStruct((LP,), jnp.float32),
            jax.ShapeDtypeStruct((LP,), jnp.float32),
        ),
        scratch_types=(
            [pltpu.VMEM((ZB,), jnp.float32),
             pltpu.VMEM((128,), jnp.float32),
             pltpu.VMEM((1024,), jnp.int32),
             pltpu.VMEM((1024,), jnp.int32)]
            + [pltpu.VMEM((128,), jnp.int32) for _ in range(16)]
            + [pltpu.SemaphoreType.DMA, pltpu.SemaphoreType.DMA]
        ),
    )(_s2_body)
    return f(se, de, rank_ext)


def _t1_body(agg0_ref, x_ref, deg_ref, w0_ref, b0_ref, p0n_ref, wu_ref,
             x0_ref, score_ref, y0_ref):
    a = (agg0_ref[...] + x_ref[...])
    x0 = jnp.maximum(jnp.dot(a, w0_ref[...], preferred_element_type=jnp.float32)
                     / deg_ref[...] + b0_ref[...], 0.0)
    x0_ref[...] = x0
    score_ref[...] = jnp.dot(x0, p0n_ref[...], preferred_element_type=jnp.float32)
    y0_ref[...] = jnp.dot(x0, wu_ref[...], preferred_element_type=jnp.float32)


def _t1(agg0, x, deg, W0, b0, p0n, Wu):
    return pl.pallas_call(
        _t1_body,
        out_shape=(
            jax.ShapeDtypeStruct((N, D), jnp.float32),
            jax.ShapeDtypeStruct((N, 1), jnp.float32),
            jax.ShapeDtypeStruct((N, 1), jnp.float32),
        ),
    )(agg0, x, deg, W0, b0, p0n, Wu)


KP = 5120
BI = 1000
BJ = 512
BT = 1024


def _t2_body(atr_ref, ar_ref, m_ref, deg1_ref):
    t = pl.program_id(1)
    j = pl.program_id(2)
    nt = pl.num_programs(1)
    part = jnp.dot(atr_ref[...].astype(jnp.bfloat16),
                   ar_ref[...].astype(jnp.bfloat16).T,
                   preferred_element_type=jnp.float32)
    js = pl.ds(j * BJ, BJ)

    @pl.when(t == 0)
    def _():
        m_ref[:, js] = part

    @pl.when(t > 0)
    def _():
        m_ref[:, js] += part

    @pl.when(t == nt - 1)
    def _():
        m = (m_ref[:, js] > 0.0).astype(jnp.float32)
        m_ref[:, js] = m

        @pl.when(j == 0)
        def _():
            deg1_ref[...] = jnp.zeros_like(deg1_ref)

        cols = j * BJ + lax.broadcasted_iota(jnp.int32, (1, BJ), 1)
        m_deg = jnp.where(cols < K, m, 0.0)
        deg1_ref[...] += jnp.sum(m_deg, axis=1, keepdims=True)


def _t2(ATr, Ar_pad):
    grid = (K // BI, NP // BT, KP // BJ)
    return pl.pallas_call(
        _t2_body,
        grid=grid,
        in_specs=[
            pl.BlockSpec((BI, BT), lambda i, t, j: (i, t)),
            pl.BlockSpec((BJ, BT), lambda i, t, j: (j, t)),
        ],
        out_specs=(
            pl.BlockSpec((BI, KP), lambda i, t, j: (i, 0)),
            pl.BlockSpec((BI, 1), lambda i, t, j: (i, 0)),
        ),
        out_shape=(
            jax.ShapeDtypeStruct((K, KP), jnp.float32),
            jax.ShapeDtypeStruct((K, 1), jnp.float32),
        ),
        compiler_params=pltpu.CompilerParams(
            dimension_semantics=("parallel", "arbitrary", "arbitrary")),
    )(ATr, Ar_pad)


def _t3_body(xpr_ref, sp_ref, w1_ref, out_ref):
    xp = xpr_ref[...] * jnp.tanh(sp_ref[...])
    out_ref[...] = jnp.dot(xp, w1_ref[...], preferred_element_type=jnp.float32)


def _t3(xp_rows, sp, W1):
    return pl.pallas_call(
        _t3_body,
        out_shape=jax.ShapeDtypeStruct((K, D), jnp.float32),
    )(xp_rows, sp, W1)


BI4 = 200


def _t4_body(m_ref, xw1_ref, deg1_ref, b1_ref, wu_ref, y1_ref):
    agg = jnp.dot(m_ref[...].astype(jnp.float32), xw1_ref[...],
                  preferred_element_type=jnp.float32)
    x1 = jnp.maximum(agg / jnp.maximum(deg1_ref[...], 1.0) + b1_ref[...], 0.0)
    y1_ref[...] = jnp.dot(x1, wu_ref[...], preferred_element_type=jnp.float32)


def _t4(M, xw1, deg1, b1, Wu):
    grid = (K // BI4,)
    return pl.pallas_call(
        _t4_body,
        grid=grid,
        in_specs=[
            pl.BlockSpec((BI4, KP), lambda i: (i, 0)),
            pl.BlockSpec((KP, D), lambda i: (0, 0)),
            pl.BlockSpec((BI4, 1), lambda i: (i, 0)),
            pl.BlockSpec((1, D), lambda i: (0, 0)),
            pl.BlockSpec((D, 1), lambda i: (0, 0)),
        ],
        out_specs=pl.BlockSpec((BI4, 1), lambda i: (i, 0)),
        out_shape=jax.ShapeDtypeStruct((K, 1), jnp.float32),
    )(M, xw1, deg1, b1, Wu)


def _t6_body(aggu_ref, y_ref, deg_ref, bu_ref, lab_ref, xo_ref, loss_ref):
    xo = (aggu_ref[...] + y_ref[...]) / deg_ref[...] + bu_ref[...]
    xo_ref[...] = xo
    xcl = jnp.clip(xo, 1e-7, 1.0 - 1e-7)
    lab = lab_ref[...]
    pos = jnp.sum(lab)
    neg = lab.size - pos
    wgt = lab * (neg / jnp.maximum(pos, 1.0)) + (1.0 - lab)
    bce = -(lab * jnp.log(xcl) + (1.0 - lab) * jnp.log(1.0 - xcl))
    loss_ref[...] = jnp.reshape(jnp.sum(wgt * bce) / lab.size, (1, 1))


def _t6(aggu, y, deg, bu, labels):
    return pl.pallas_call(
        _t6_body,
        out_shape=(
            jax.ShapeDtypeStruct((N, 1), jnp.float32),
            jax.ShapeDtypeStruct((1, 1), jnp.float32),
        ),
    )(aggu, y, deg, bu, labels)


def kernel(x, edge_index, labels, W0, U0, c0, b0, W1, U1, c1, b1, Wu, Uu, cu, bu, p0):
    src = edge_index[0]
    dst = edge_index[1]

    agg0 = jax.ops.segment_sum(x[src], dst, num_segments=N)
    deg = (jax.ops.segment_sum(jnp.ones((E,), jnp.float32), dst, num_segments=N)
           + 1.0)[:, None]

    p0n = (p0 / jnp.linalg.norm(p0))[:, None]
    x0, score2, y0 = _t1(agg0, x, deg, W0, b0[None, :], p0n, Wu)
    score = score2[:, 0]

    perm = jax.lax.top_k(score, K)[1]

    rank = jnp.full((N,), K, jnp.int32).at[perm].set(
        jnp.arange(K, dtype=jnp.int32))
    rank_ext = jnp.concatenate([rank, jnp.full((NP - N,), K, jnp.int32)])
    fill = jnp.full((E_PAD - E - K,), N, jnp.int32)
    se = jnp.concatenate([src, perm.astype(jnp.int32), fill])
    de = jnp.concatenate([dst, perm.astype(jnp.int32), fill])
    ar_flat, atr_flat = _s2(se, de, rank_ext)
    Ar_full = ar_flat.reshape(KR, NP)
    ATr_full = atr_flat.reshape(KR, NP)

    M, deg1 = _t2(ATr_full, Ar_full)

    xp_rows = x0[perm]
    sp = score[perm][:, None]
    xw1 = _t3(xp_rows, sp, W1)
    xw1_pad = jnp.concatenate([xw1, jnp.zeros((KP - K, D), jnp.float32)], axis=0)
    y1 = _t4(M, xw1_pad, deg1, b1[None, :], Wu)

    y = y0.at[perm, 0].add(y1[:, 0])
    aggu = jax.ops.segment_sum(y[src, 0], dst, num_segments=N)[:, None]
    xo, loss = _t6(aggu, y, deg, bu, labels)
    return (loss[0, 0], xo)


# speedup vs baseline: 2.0385x; 1.2331x over previous
import functools
import math

import jax
import jax.numpy as jnp
from jax import lax
from jax.experimental import pallas as pl
from jax.experimental.pallas import tpu as pltpu
from jax.experimental.pallas import tpu_sc as plsc

N = 10000
E = 320000
D = 128
K = 5000

NP = 10240
KR = 5120
LP = KR * NP
E_PAD = 327680
EPT = E_PAD // 16
ZSPAN = LP // 16
ZB = 16384
ZITERS = ZSPAN // ZB


def _s2_body(se_hbm, de_hbm, rank_hbm, ar_hbm, atr_hbm,
             zeros_v, ones_v, se_v, de_v,
             i0, i1, i2, i3, i4, i5, i6, i7,
             r0, r1, r2, r3, r4, r5, r6, r7, sem, sem2):
    c_ax = lax.axis_index("c")
    s_ax = lax.axis_index("s")
    idxrs = (i0, i1, i2, i3, i4, i5, i6, i7)
    rks = (r0, r1, r2, r3, r4, r5, r6, r7)

    def init_bufs(i, _):
        zeros_v[pl.ds(i * 16, 16)] = jnp.zeros((16,), jnp.float32)
        return 0

    lax.fori_loop(0, ZB // 16, init_bufs, 0)
    for j in range(8):
        ones_v[pl.ds(j * 16, 16)] = jnp.ones((16,), jnp.float32)

    def do_zero(out_ref):
        zb = s_ax * ZSPAN

        def zstep(i, _):
            pltpu.sync_copy(zeros_v, out_ref.at[pl.ds(zb + i * ZB, ZB)])
            return 0

        lax.fori_loop(0, ZITERS, zstep, 0)

    def do_scatter(out_ref, is_ar):
        tbase = s_ax * EPT
        key_v = se_v if is_ar else de_v
        col_v = de_v if is_ar else se_v

        def ostep(o, _):
            pltpu.sync_copy(se_hbm.at[pl.ds(tbase + o * 1024, 1024)], se_v)
            pltpu.sync_copy(de_hbm.at[pl.ds(tbase + o * 1024, 1024)], de_v)
            gathers = [
                pltpu.async_copy(
                    rank_hbm.at[key_v.at[pl.ds(r * 128, 128)]], rks[r], sem2)
                for r in range(8)
            ]
            for cp in gathers:
                cp.wait()
            for r in range(8):
                for j in range(8):
                    off = r * 128 + j * 16
                    rk16 = rks[r][pl.ds(j * 16, 16)]
                    col16 = col_v[pl.ds(off, 16)]
                    idxrs[r][pl.ds(j * 16, 16)] = rk16 * NP + col16
            scatters = [
                pltpu.async_copy(ones_v, out_ref.at[idxrs[r]], sem)
                for r in range(8)
            ]
            for cp in scatters:
                cp.wait()
            return 0

        lax.fori_loop(0, EPT // 1024, ostep, 0)

    @pl.when(c_ax == 0)
    def _():
        do_zero(ar_hbm)

    @pl.when(c_ax == 1)
    def _():
        do_zero(atr_hbm)

    plsc.subcore_barrier()

    @pl.when(c_ax == 0)
    def _():
        do_scatter(ar_hbm, True)

    @pl.when(c_ax == 1)
    def _():
        do_scatter(atr_hbm, False)


def _s2(se, de, rank_ext):
    mesh = plsc.VectorSubcoreMesh(core_axis_name="c", subcore_axis_name="s")
    f = functools.partial(
        pl.kernel, mesh=mesh,
        out_type=(
            jax.ShapeDtypeStruct((LP,), jnp.float32),
            jax.ShapeDtypeStruct((LP,), jnp.float32),
        ),
        scratch_types=(
            [pltpu.VMEM((ZB,), jnp.float32),
             pltpu.VMEM((128,), jnp.float32),
             pltpu.VMEM((1024,), jnp.int32),
             pltpu.VMEM((1024,), jnp.int32)]
            + [pltpu.VMEM((128,), jnp.int32) for _ in range(16)]
            + [pltpu.SemaphoreType.DMA, pltpu.SemaphoreType.DMA]
        ),
    )(_s2_body)
    return f(se, de, rank_ext)


NACC = 10016
EPW = E_PAD // 32
RING = 2


def _s1_body(x_hbm, se_hbm, de_hbm, agg_hbm, deg_hbm,
             acc_sp, deg_sp,
             zcol, ones_v, se_v, de_v,
             w0, w1, x0_, x1_,
             semg, sems):
    c_ax = lax.axis_index("c")
    s_ax = lax.axis_index("s")
    rows = (w0, w1)
    didx = (x0_, x1_)

    def initz(i, _):
        zcol[pl.ds(i * 16, 16)] = jnp.zeros((16,), jnp.float32)
        return 0

    lax.fori_loop(0, 40, initz, 0)
    for j in range(8):
        ones_v[pl.ds(j * 16, 16)] = jnp.ones((16,), jnp.float32)

    def initzr2(i, _):
        r = i // 8
        cc = (i % 8) * 16
        w0[r, pl.ds(cc, 16)] = jnp.zeros((16,), jnp.float32)
        return 0

    lax.fori_loop(0, 128 * 8, initzr2, 0)

    r0 = s_ax * 624
    for q, nr in ((0, 128), (128, 128), (256, 128), (384, 128), (512, 112)):
        pltpu.sync_copy(w0.at[pl.ds(0, nr), :],
                        acc_sp.at[pl.ds(r0 + q, nr), :])

    @pl.when(s_ax == 0)
    def _():
        pltpu.sync_copy(w0.at[pl.ds(0, 32), :],
                        acc_sp.at[pl.ds(9984, 32), :])

    pltpu.sync_copy(zcol, deg_sp.at[pl.ds(s_ax * 640, 640)])
    plsc.subcore_barrier()

    wid = c_ax * 16 + s_ax
    ebase = wid * EPW

    def build_didx(c):
        for j in range(8):
            didx[c % RING][pl.ds(j * 16, 16)] = de_v[pl.ds(c * 128 + j * 16, 16)]

    def fire_gather(c):
        return pltpu.async_copy(
            x_hbm.at[se_v.at[pl.ds(c * 128, 128)]], rows[c % RING], semg)

    def ostep(o, _):
        pltpu.sync_copy(se_hbm.at[pl.ds(ebase + o * 1280, 1280)], se_v)
        pltpu.sync_copy(de_hbm.at[pl.ds(ebase + o * 1280, 1280)], de_v)
        g = {}
        s = {}
        g[0] = fire_gather(0)
        for c in range(10):
            g[c].wait()
            build_didx(c)
            s[c] = (
                pltpu.async_copy(rows[c % RING],
                                 acc_sp.at[didx[c % RING]], sems, add=True),
                pltpu.async_copy(ones_v,
                                 deg_sp.at[didx[c % RING]], sems, add=True),
            )
            if c + 1 < 10:
                if c >= 1:
                    s[c - 1][0].wait()
                    s[c - 1][1].wait()
                g[c + 1] = fire_gather(c + 1)
        for c in range(8, 10):
            s[c][0].wait()
            s[c][1].wait()
        return 0

    lax.fori_loop(0, EPW // 1280, ostep, 0)
    plsc.subcore_barrier()

    pltpu.sync_copy(acc_sp.at[pl.ds(s_ax * 624, 624), :],
                    agg_hbm.at[c_ax, pl.ds(s_ax * 624, 624), :])

    @pl.when(s_ax == 0)
    def _():
        pltpu.sync_copy(acc_sp.at[pl.ds(9984, 16), :],
                        agg_hbm.at[c_ax, pl.ds(9984, 16), :])

    pltpu.sync_copy(deg_sp.at[pl.ds(s_ax * 640, 640)],
                    deg_hbm.at[c_ax, pl.ds(s_ax * 640, 640)])


def _s1(x, se, de):
    mesh = plsc.VectorSubcoreMesh(core_axis_name="c", subcore_axis_name="s")
    f = functools.partial(
        pl.kernel, mesh=mesh,
        out_type=(
            jax.ShapeDtypeStruct((2, N, D), jnp.float32),
            jax.ShapeDtypeStruct((2, NP), jnp.float32),
        ),
        scratch_types=(
            [pltpu.VMEM_SHARED((NACC, D), jnp.float32),
             pltpu.VMEM_SHARED((NP,), jnp.float32),
             pltpu.VMEM((640,), jnp.float32),
             pltpu.VMEM((128,), jnp.float32),
             pltpu.VMEM((1280,), jnp.int32),
             pltpu.VMEM((1280,), jnp.int32)]
            + [pltpu.VMEM((128, D), jnp.float32) for _ in range(RING)]
            + [pltpu.VMEM((128,), jnp.int32) for _ in range(RING)]
            + [pltpu.SemaphoreType.DMA, pltpu.SemaphoreType.DMA]
        ),
    )(_s1_body)
    return f(x, se, de)


def _t1_body(agg0_ref, x_ref, deg_ref, w0_ref, b0_ref, p0n_ref, wu_ref,
             x0_ref, score_ref, y0_ref):
    a = (agg0_ref[0] + agg0_ref[1] + x_ref[...])
    x0 = jnp.maximum(jnp.dot(a, w0_ref[...], preferred_element_type=jnp.float32)
                     / deg_ref[...] + b0_ref[...], 0.0)
    x0_ref[...] = x0
    score_ref[...] = jnp.dot(x0, p0n_ref[...], preferred_element_type=jnp.float32)
    y0_ref[...] = jnp.dot(x0, wu_ref[...], preferred_element_type=jnp.float32)


def _t1(agg0, x, deg, W0, b0, p0n, Wu):
    return pl.pallas_call(
        _t1_body,
        out_shape=(
            jax.ShapeDtypeStruct((N, D), jnp.float32),
            jax.ShapeDtypeStruct((N, 1), jnp.float32),
            jax.ShapeDtypeStruct((N, 1), jnp.float32),
        ),
    )(agg0, x, deg, W0, b0, p0n, Wu)


KP = 5120
BI = 1000
BJ = 512
BT = 1024


def _t2_body(atr_ref, ar_ref, m_ref, deg1_ref):
    t = pl.program_id(1)
    j = pl.program_id(2)
    nt = pl.num_programs(1)
    part = jnp.dot(atr_ref[...].astype(jnp.bfloat16),
                   ar_ref[...].astype(jnp.bfloat16).T,
                   preferred_element_type=jnp.float32)
    js = pl.ds(j * BJ, BJ)

    @pl.when(t == 0)
    def _():
        m_ref[:, js] = part

    @pl.when(t > 0)
    def _():
        m_ref[:, js] += part

    @pl.when(t == nt - 1)
    def _():
        m = (m_ref[:, js] > 0.0).astype(jnp.float32)
        m_ref[:, js] = m

        @pl.when(j == 0)
        def _():
            deg1_ref[...] = jnp.zeros_like(deg1_ref)

        cols = j * BJ + lax.broadcasted_iota(jnp.int32, (1, BJ), 1)
        m_deg = jnp.where(cols < K, m, 0.0)
        deg1_ref[...] += jnp.sum(m_deg, axis=1, keepdims=True)


def _t2(ATr, Ar_pad):
    grid = (K // BI, NP // BT, KP // BJ)
    return pl.pallas_call(
        _t2_body,
        grid=grid,
        in_specs=[
            pl.BlockSpec((BI, BT), lambda i, t, j: (i, t)),
            pl.BlockSpec((BJ, BT), lambda i, t, j: (j, t)),
        ],
        out_specs=(
            pl.BlockSpec((BI, KP), lambda i, t, j: (i, 0)),
            pl.BlockSpec((BI, 1), lambda i, t, j: (i, 0)),
        ),
        out_shape=(
            jax.ShapeDtypeStruct((K, KP), jnp.float32),
            jax.ShapeDtypeStruct((K, 1), jnp.float32),
        ),
        compiler_params=pltpu.CompilerParams(
            dimension_semantics=("parallel", "arbitrary", "arbitrary")),
    )(ATr, Ar_pad)


def _t3_body(xpr_ref, sp_ref, w1_ref, out_ref):
    xp = xpr_ref[...] * jnp.tanh(sp_ref[...])
    out_ref[...] = jnp.dot(xp, w1_ref[...], preferred_element_type=jnp.float32)


def _t3(xp_rows, sp, W1):
    return pl.pallas_call(
        _t3_body,
        out_shape=jax.ShapeDtypeStruct((K, D), jnp.float32),
    )(xp_rows, sp, W1)


BI4 = 200


def _t4_body(m_ref, xw1_ref, deg1_ref, b1_ref, wu_ref, y1_ref):
    agg = jnp.dot(m_ref[...].astype(jnp.float32), xw1_ref[...],
                  preferred_element_type=jnp.float32)
    x1 = jnp.maximum(agg / jnp.maximum(deg1_ref[...], 1.0) + b1_ref[...], 0.0)
    y1_ref[...] = jnp.dot(x1, wu_ref[...], preferred_element_type=jnp.float32)


def _t4(M, xw1, deg1, b1, Wu):
    grid = (K // BI4,)
    return pl.pallas_call(
        _t4_body,
        grid=grid,
        in_specs=[
            pl.BlockSpec((BI4, KP), lambda i: (i, 0)),
            pl.BlockSpec((KP, D), lambda i: (0, 0)),
            pl.BlockSpec((BI4, 1), lambda i: (i, 0)),
            pl.BlockSpec((1, D), lambda i: (0, 0)),
            pl.BlockSpec((D, 1), lambda i: (0, 0)),
        ],
        out_specs=pl.BlockSpec((BI4, 1), lambda i: (i, 0)),
        out_shape=jax.ShapeDtypeStruct((K, 1), jnp.float32),
    )(M, xw1, deg1, b1, Wu)


def _t6_body(aggu_ref, y_ref, deg_ref, bu_ref, lab_ref, xo_ref, loss_ref):
    xo = (aggu_ref[...] + y_ref[...]) / deg_ref[...] + bu_ref[...]
    xo_ref[...] = xo
    xcl = jnp.clip(xo, 1e-7, 1.0 - 1e-7)
    lab = lab_ref[...]
    pos = jnp.sum(lab)
    neg = lab.size - pos
    wgt = lab * (neg / jnp.maximum(pos, 1.0)) + (1.0 - lab)
    bce = -(lab * jnp.log(xcl) + (1.0 - lab) * jnp.log(1.0 - xcl))
    loss_ref[...] = jnp.reshape(jnp.sum(wgt * bce) / lab.size, (1, 1))


def _t6(aggu, y, deg, bu, labels):
    return pl.pallas_call(
        _t6_body,
        out_shape=(
            jax.ShapeDtypeStruct((N, 1), jnp.float32),
            jax.ShapeDtypeStruct((1, 1), jnp.float32),
        ),
    )(aggu, y, deg, bu, labels)


def kernel(x, edge_index, labels, W0, U0, c0, b0, W1, U1, c1, b1, Wu, Uu, cu, bu, p0):
    src = edge_index[0]
    dst = edge_index[1]

    fill_s = jnp.zeros((E_PAD - E,), jnp.int32)
    fill_d = jnp.full((E_PAD - E,), N, jnp.int32)
    se1 = jnp.concatenate([src, fill_s])
    de1 = jnp.concatenate([dst, fill_d])
    aggp, degp = _s1(x, se1, de1)
    deg = (degp[0] + degp[1] + 1.0)[:N, None]

    p0n = (p0 / jnp.linalg.norm(p0))[:, None]
    x0, score2, y0 = _t1(aggp, x, deg, W0, b0[None, :], p0n, Wu)
    score = score2[:, 0]

    perm = jax.lax.top_k(score, K)[1]

    rank = jnp.full((N,), K, jnp.int32).at[perm].set(
        jnp.arange(K, dtype=jnp.int32))
    rank_ext = jnp.concatenate([rank, jnp.full((NP - N,), K, jnp.int32)])
    fill = jnp.full((E_PAD - E - K,), N, jnp.int32)
    se = jnp.concatenate([src, perm.astype(jnp.int32), fill])
    de = jnp.concatenate([dst, perm.astype(jnp.int32), fill])
    ar_flat, atr_flat = _s2(se, de, rank_ext)
    Ar_full = ar_flat.reshape(KR, NP)
    ATr_full = atr_flat.reshape(KR, NP)

    M, deg1 = _t2(ATr_full, Ar_full)

    xp_rows = x0[perm]
    sp = score[perm][:, None]
    xw1 = _t3(xp_rows, sp, W1)
    xw1_pad = jnp.concatenate([xw1, jnp.zeros((KP - K, D), jnp.float32)], axis=0)
    y1 = _t4(M, xw1_pad, deg1, b1[None, :], Wu)

    y = y0.at[perm, 0].add(y1[:, 0])
    aggu = jax.ops.segment_sum(y[src, 0], dst, num_segments=N)[:, None]
    xo, loss = _t6(aggu, y, deg, bu, labels)
    return (loss[0, 0], xo)


# speedup vs baseline: 2.0393x; 1.0004x over previous
import functools
import math

import jax
import jax.numpy as jnp
from jax import lax
from jax.experimental import pallas as pl
from jax.experimental.pallas import tpu as pltpu
from jax.experimental.pallas import tpu_sc as plsc

N = 10000
E = 320000
D = 128
K = 5000

NP = 10240
KR = 5120
LP = KR * NP
E_PAD = 327680
EPT = E_PAD // 16
ZSPAN = LP // 16
ZB = 16384
ZITERS = ZSPAN // ZB


def _s2_body(se_hbm, de_hbm, rank_hbm, ar_hbm, atr_hbm,
             zeros_v, ones_v, se_v, de_v,
             i0, i1, i2, i3, i4, i5, i6, i7,
             r0, r1, r2, r3, r4, r5, r6, r7, sem, sem2):
    c_ax = lax.axis_index("c")
    s_ax = lax.axis_index("s")
    idxrs = (i0, i1, i2, i3, i4, i5, i6, i7)
    rks = (r0, r1, r2, r3, r4, r5, r6, r7)

    def init_bufs(i, _):
        zeros_v[pl.ds(i * 16, 16)] = jnp.zeros((16,), jnp.float32)
        return 0

    lax.fori_loop(0, ZB // 16, init_bufs, 0)
    for j in range(8):
        ones_v[pl.ds(j * 16, 16)] = jnp.ones((16,), jnp.float32)

    def do_zero(out_ref):
        zb = s_ax * ZSPAN

        def zstep(i, _):
            cps = [
                pltpu.async_copy(
                    zeros_v, out_ref.at[pl.ds(zb + (i * 8 + r) * ZB, ZB)], sem2)
                for r in range(8)
            ]
            for cp in cps:
                cp.wait()
            return 0

        lax.fori_loop(0, ZITERS // 8, zstep, 0)

    def do_scatter(out_ref, is_ar):
        tbase = s_ax * EPT
        key_v = se_v if is_ar else de_v
        col_v = de_v if is_ar else se_v

        def ostep(o, _):
            pltpu.sync_copy(se_hbm.at[pl.ds(tbase + o * 1024, 1024)], se_v)
            pltpu.sync_copy(de_hbm.at[pl.ds(tbase + o * 1024, 1024)], de_v)
            gathers = [
                pltpu.async_copy(
                    rank_hbm.at[key_v.at[pl.ds(r * 128, 128)]], rks[r], sem2)
                for r in range(8)
            ]
            for cp in gathers:
                cp.wait()
            for r in range(8):
                for j in range(8):
                    off = r * 128 + j * 16
                    rk16 = rks[r][pl.ds(j * 16, 16)]
                    col16 = col_v[pl.ds(off, 16)]
                    idxrs[r][pl.ds(j * 16, 16)] = rk16 * NP + col16
            scatters = [
                pltpu.async_copy(ones_v, out_ref.at[idxrs[r]], sem)
                for r in range(8)
            ]
            for cp in scatters:
                cp.wait()
            return 0

        lax.fori_loop(0, EPT // 1024, ostep, 0)

    @pl.when(c_ax == 0)
    def _():
        do_zero(ar_hbm)

    @pl.when(c_ax == 1)
    def _():
        do_zero(atr_hbm)

    plsc.subcore_barrier()

    @pl.when(c_ax == 0)
    def _():
        do_scatter(ar_hbm, True)

    @pl.when(c_ax == 1)
    def _():
        do_scatter(atr_hbm, False)


def _s2(se, de, rank_ext):
    mesh = plsc.VectorSubcoreMesh(core_axis_name="c", subcore_axis_name="s")
    f = functools.partial(
        pl.kernel, mesh=mesh,
        out_type=(
            jax.ShapeDtypeStruct((LP,), jnp.float32),
            jax.ShapeDtypeStruct((LP,), jnp.float32),
        ),
        scratch_types=(
            [pltpu.VMEM((ZB,), jnp.float32),
             pltpu.VMEM((128,), jnp.float32),
             pltpu.VMEM((1024,), jnp.int32),
             pltpu.VMEM((1024,), jnp.int32)]
            + [pltpu.VMEM((128,), jnp.int32) for _ in range(16)]
            + [pltpu.SemaphoreType.DMA, pltpu.SemaphoreType.DMA]
        ),
    )(_s2_body)
    return f(se, de, rank_ext)


NACC = 10016
EPW = E_PAD // 32
RING = 2


def _s1_body(x_hbm, se_hbm, de_hbm, agg_hbm, deg_hbm,
             acc_sp, deg_sp,
             zcol, ones_v, se_v, de_v,
             w0, w1, x0_, x1_,
             semg, sems):
    c_ax = lax.axis_index("c")
    s_ax = lax.axis_index("s")
    rows = (w0, w1)
    didx = (x0_, x1_)

    def initz(i, _):
        zcol[pl.ds(i * 16, 16)] = jnp.zeros((16,), jnp.float32)
        return 0

    lax.fori_loop(0, 40, initz, 0)
    for j in range(8):
        ones_v[pl.ds(j * 16, 16)] = jnp.ones((16,), jnp.float32)

    def initzr2(i, _):
        r = i // 8
        cc = (i % 8) * 16
        w0[r, pl.ds(cc, 16)] = jnp.zeros((16,), jnp.float32)
        return 0

    lax.fori_loop(0, 128 * 8, initzr2, 0)

    r0 = s_ax * 624
    for q, nr in ((0, 128), (128, 128), (256, 128), (384, 128), (512, 112)):
        pltpu.sync_copy(w0.at[pl.ds(0, nr), :],
                        acc_sp.at[pl.ds(r0 + q, nr), :])

    @pl.when(s_ax == 0)
    def _():
        pltpu.sync_copy(w0.at[pl.ds(0, 32), :],
                        acc_sp.at[pl.ds(9984, 32), :])

    pltpu.sync_copy(zcol, deg_sp.at[pl.ds(s_ax * 640, 640)])
    plsc.subcore_barrier()

    wid = c_ax * 16 + s_ax
    ebase = wid * EPW

    def build_didx(c):
        for j in range(8):
            didx[c % RING][pl.ds(j * 16, 16)] = de_v[pl.ds(c * 128 + j * 16, 16)]

    def fire_gather(c):
        return pltpu.async_copy(
            x_hbm.at[se_v.at[pl.ds(c * 128, 128)]], rows[c % RING], semg)

    def ostep(o, _):
        pltpu.sync_copy(se_hbm.at[pl.ds(ebase + o * 1280, 1280)], se_v)
        pltpu.sync_copy(de_hbm.at[pl.ds(ebase + o * 1280, 1280)], de_v)
        g = {}
        s = {}
        g[0] = fire_gather(0)
        for c in range(10):
            g[c].wait()
            build_didx(c)
            s[c] = (
                pltpu.async_copy(rows[c % RING],
                                 acc_sp.at[didx[c % RING]], sems, add=True),
                pltpu.async_copy(ones_v,
                                 deg_sp.at[didx[c % RING]], sems, add=True),
            )
            if c + 1 < 10:
                if c >= 1:
                    s[c - 1][0].wait()
                    s[c - 1][1].wait()
                g[c + 1] = fire_gather(c + 1)
        for c in range(8, 10):
            s[c][0].wait()
            s[c][1].wait()
        return 0

    lax.fori_loop(0, EPW // 1280, ostep, 0)
    plsc.subcore_barrier()

    pltpu.sync_copy(acc_sp.at[pl.ds(s_ax * 624, 624), :],
                    agg_hbm.at[c_ax, pl.ds(s_ax * 624, 624), :])

    @pl.when(s_ax == 0)
    def _():
        pltpu.sync_copy(acc_sp.at[pl.ds(9984, 16), :],
                        agg_hbm.at[c_ax, pl.ds(9984, 16), :])

    pltpu.sync_copy(deg_sp.at[pl.ds(s_ax * 640, 640)],
                    deg_hbm.at[c_ax, pl.ds(s_ax * 640, 640)])


def _s1(x, se, de):
    mesh = plsc.VectorSubcoreMesh(core_axis_name="c", subcore_axis_name="s")
    f = functools.partial(
        pl.kernel, mesh=mesh,
        out_type=(
            jax.ShapeDtypeStruct((2, N, D), jnp.float32),
            jax.ShapeDtypeStruct((2, NP), jnp.float32),
        ),
        scratch_types=(
            [pltpu.VMEM_SHARED((NACC, D), jnp.float32),
             pltpu.VMEM_SHARED((NP,), jnp.float32),
             pltpu.VMEM((640,), jnp.float32),
             pltpu.VMEM((128,), jnp.float32),
             pltpu.VMEM((1280,), jnp.int32),
             pltpu.VMEM((1280,), jnp.int32)]
            + [pltpu.VMEM((128, D), jnp.float32) for _ in range(RING)]
            + [pltpu.VMEM((128,), jnp.int32) for _ in range(RING)]
            + [pltpu.SemaphoreType.DMA, pltpu.SemaphoreType.DMA]
        ),
    )(_s1_body)
    return f(x, se, de)


def _t1_body(agg0_ref, x_ref, deg_ref, w0_ref, b0_ref, p0n_ref, wu_ref,
             x0_ref, score_ref, y0_ref):
    a = (agg0_ref[0] + agg0_ref[1] + x_ref[...])
    x0 = jnp.maximum(jnp.dot(a, w0_ref[...], preferred_element_type=jnp.float32)
                     / deg_ref[...] + b0_ref[...], 0.0)
    x0_ref[...] = x0
    score_ref[...] = jnp.dot(x0, p0n_ref[...], preferred_element_type=jnp.float32)
    y0_ref[...] = jnp.dot(x0, wu_ref[...], preferred_element_type=jnp.float32)


def _t1(agg0, x, deg, W0, b0, p0n, Wu):
    return pl.pallas_call(
        _t1_body,
        out_shape=(
            jax.ShapeDtypeStruct((N, D), jnp.float32),
            jax.ShapeDtypeStruct((N, 1), jnp.float32),
            jax.ShapeDtypeStruct((N, 1), jnp.float32),
        ),
    )(agg0, x, deg, W0, b0, p0n, Wu)


KP = 5120
BI = 1000
BJ = 512
BT = 1024


def _t2_body(atr_ref, ar_ref, m_ref, deg1_ref):
    t = pl.program_id(1)
    j = pl.program_id(2)
    nt = pl.num_programs(1)
    part = jnp.dot(atr_ref[...].astype(jnp.bfloat16),
                   ar_ref[...].astype(jnp.bfloat16).T,
                   preferred_element_type=jnp.float32)
    js = pl.ds(j * BJ, BJ)

    @pl.when(t == 0)
    def _():
        m_ref[:, js] = part

    @pl.when(t > 0)
    def _():
        m_ref[:, js] += part

    @pl.when(t == nt - 1)
    def _():
        m = (m_ref[:, js] > 0.0).astype(jnp.float32)
        m_ref[:, js] = m

        @pl.when(j == 0)
        def _():
            deg1_ref[...] = jnp.zeros_like(deg1_ref)

        cols = j * BJ + lax.broadcasted_iota(jnp.int32, (1, BJ), 1)
        m_deg = jnp.where(cols < K, m, 0.0)
        deg1_ref[...] += jnp.sum(m_deg, axis=1, keepdims=True)


def _t2(ATr, Ar_pad):
    grid = (K // BI, NP // BT, KP // BJ)
    return pl.pallas_call(
        _t2_body,
        grid=grid,
        in_specs=[
            pl.BlockSpec((BI, BT), lambda i, t, j: (i, t)),
            pl.BlockSpec((BJ, BT), lambda i, t, j: (j, t)),
        ],
        out_specs=(
            pl.BlockSpec((BI, KP), lambda i, t, j: (i, 0)),
            pl.BlockSpec((BI, 1), lambda i, t, j: (i, 0)),
        ),
        out_shape=(
            jax.ShapeDtypeStruct((K, KP), jnp.float32),
            jax.ShapeDtypeStruct((K, 1), jnp.float32),
        ),
        compiler_params=pltpu.CompilerParams(
            dimension_semantics=("parallel", "arbitrary", "arbitrary")),
    )(ATr, Ar_pad)


def _t3_body(xpr_ref, sp_ref, w1_ref, out_ref):
    xp = xpr_ref[...] * jnp.tanh(sp_ref[...])
    out_ref[...] = jnp.dot(xp, w1_ref[...], preferred_element_type=jnp.float32)


def _t3(xp_rows, sp, W1):
    return pl.pallas_call(
        _t3_body,
        out_shape=jax.ShapeDtypeStruct((K, D), jnp.float32),
    )(xp_rows, sp, W1)


BI4 = 200


def _t4_body(m_ref, xw1_ref, deg1_ref, b1_ref, wu_ref, y1_ref):
    agg = jnp.dot(m_ref[...].astype(jnp.float32), xw1_ref[...],
                  preferred_element_type=jnp.float32)
    x1 = jnp.maximum(agg / jnp.maximum(deg1_ref[...], 1.0) + b1_ref[...], 0.0)
    y1_ref[...] = jnp.dot(x1, wu_ref[...], preferred_element_type=jnp.float32)


def _t4(M, xw1, deg1, b1, Wu):
    grid = (K // BI4,)
    return pl.pallas_call(
        _t4_body,
        grid=grid,
        in_specs=[
            pl.BlockSpec((BI4, KP), lambda i: (i, 0)),
            pl.BlockSpec((KP, D), lambda i: (0, 0)),
            pl.BlockSpec((BI4, 1), lambda i: (i, 0)),
            pl.BlockSpec((1, D), lambda i: (0, 0)),
            pl.BlockSpec((D, 1), lambda i: (0, 0)),
        ],
        out_specs=pl.BlockSpec((BI4, 1), lambda i: (i, 0)),
        out_shape=jax.ShapeDtypeStruct((K, 1), jnp.float32),
    )(M, xw1, deg1, b1, Wu)


def _t6_body(aggu_ref, y_ref, deg_ref, bu_ref, lab_ref, xo_ref, loss_ref):
    xo = (aggu_ref[...] + y_ref[...]) / deg_ref[...] + bu_ref[...]
    xo_ref[...] = xo
    xcl = jnp.clip(xo, 1e-7, 1.0 - 1e-7)
    lab = lab_ref[...]
    pos = jnp.sum(lab)
    neg = lab.size - pos
    wgt = lab * (neg / jnp.maximum(pos, 1.0)) + (1.0 - lab)
    bce = -(lab * jnp.log(xcl) + (1.0 - lab) * jnp.log(1.0 - xcl))
    loss_ref[...] = jnp.reshape(jnp.sum(wgt * bce) / lab.size, (1, 1))


def _t6(aggu, y, deg, bu, labels):
    return pl.pallas_call(
        _t6_body,
        out_shape=(
            jax.ShapeDtypeStruct((N, 1), jnp.float32),
            jax.ShapeDtypeStruct((1, 1), jnp.float32),
        ),
    )(aggu, y, deg, bu, labels)


def kernel(x, edge_index, labels, W0, U0, c0, b0, W1, U1, c1, b1, Wu, Uu, cu, bu, p0):
    src = edge_index[0]
    dst = edge_index[1]

    fill_s = jnp.zeros((E_PAD - E,), jnp.int32)
    fill_d = jnp.full((E_PAD - E,), N, jnp.int32)
    se1 = jnp.concatenate([src, fill_s])
    de1 = jnp.concatenate([dst, fill_d])
    aggp, degp = _s1(x, se1, de1)
    deg = (degp[0] + degp[1] + 1.0)[:N, None]

    p0n = (p0 / jnp.linalg.norm(p0))[:, None]
    x0, score2, y0 = _t1(aggp, x, deg, W0, b0[None, :], p0n, Wu)
    score = score2[:, 0]

    perm = jax.lax.top_k(score, K)[1]

    rank = jnp.full((N,), K, jnp.int32).at[perm].set(
        jnp.arange(K, dtype=jnp.int32))
    rank_ext = jnp.concatenate([rank, jnp.full((NP - N,), K, jnp.int32)])
    fill = jnp.full((E_PAD - E - K,), N, jnp.int32)
    se = jnp.concatenate([src, perm.astype(jnp.int32), fill])
    de = jnp.concatenate([dst, perm.astype(jnp.int32), fill])
    ar_flat, atr_flat = _s2(se, de, rank_ext)
    Ar_full = ar_flat.reshape(KR, NP)
    ATr_full = atr_flat.reshape(KR, NP)

    M, deg1 = _t2(ATr_full, Ar_full)

    xp_rows = x0[perm]
    sp = score[perm][:, None]
    xw1 = _t3(xp_rows, sp, W1)
    xw1_pad = jnp.concatenate([xw1, jnp.zeros((KP - K, D), jnp.float32)], axis=0)
    y1 = _t4(M, xw1_pad, deg1, b1[None, :], Wu)

    y = y0.at[perm, 0].add(y1[:, 0])
    aggu = jax.ops.segment_sum(y[src, 0], dst, num_segments=N)[:, None]
    xo, loss = _t6(aggu, y, deg, bu, labels)
    return (loss[0, 0], xo)
